# baseline (speedup 1.0000x reference)
import functools

import numpy as np
import jax
import jax.numpy as jnp
from jax import lax
from jax.experimental import pallas as pl
from jax.experimental.pallas import tpu as pltpu
from jax.experimental.pallas import tpu_sc as plsc


def _resize_matrix(in_size: int, out_size: int) -> np.ndarray:
    sample_f = (np.arange(out_size) + 0.5) * (in_size / out_size) - 0.5
    x = np.abs(sample_f[:, None] - np.arange(in_size)[None, :])
    w = np.maximum(0.0, 1.0 - x)
    w = w / w.sum(axis=1, keepdims=True)
    return w.astype(np.float32)


_L = 16


def _sc_indices(lr_doy, hr_doy, target_doy):
    B, Tl = lr_doy.shape
    Th = hr_doy.shape[1]
    Tt = target_doy.shape[0]
    Tlp = ((Tl + _L - 1) // _L) * _L
    Thp = ((Th + _L - 1) // _L) * _L
    Ttp = ((Tt + _L - 1) // _L) * _L
    nl, nh, nt = Tlp // _L, Thp // _L, Ttp // _L

    big = jnp.int32(2 ** 30)
    lr_flat = jnp.concatenate(
        [lr_doy, jnp.full((B, Tlp - Tl), big, jnp.int32)], axis=1).reshape(-1)
    hr_flat = jnp.concatenate(
        [hr_doy, jnp.full((B, Thp - Th), big, jnp.int32)], axis=1).reshape(-1)
    tgt_pad = jnp.concatenate(
        [target_doy, jnp.zeros((Ttp - Tt,), jnp.int32)])

    mesh = plsc.VectorSubcoreMesh(core_axis_name="c", subcore_axis_name="s")
    nc = mesh.num_cores

    def body(lr_hbm, hr_hbm, tgt_hbm, lre_hbm, lro_hbm, hre_hbm, hro_hbm,
             d_v, tgt_v, ev_v, od_v):
        wid = lax.axis_index("s") * nc + lax.axis_index("c")

        def bcast(vec, k):
            idx = jnp.full((_L, 1), k, jnp.int32)
            return lax.gather(
                vec, idx,
                lax.GatherDimensionNumbers(offset_dims=(),
                                           collapsed_slice_dims=(0,),
                                           start_index_map=(0,)),
                slice_sizes=(1,),
                mode=lax.GatherScatterMode.PROMISE_IN_BOUNDS)

        def series(d_hbm, nchunk, T, e_hbm, o_hbm, b):
            pltpu.sync_copy(d_hbm.at[pl.ds(b * nchunk * _L, nchunk * _L)],
                            d_v.at[pl.ds(0, nchunk * _L)])
            pltpu.sync_copy(tgt_hbm, tgt_v)
            one = jnp.full((_L,), 1, jnp.int32)
            zero = jnp.full((_L,), 0, jnp.int32)
            for ch in range(nt):
                t_vec = tgt_v[pl.ds(ch * _L, _L)]
                cnt = zero
                for dch in range(nchunk):
                    d_vec = d_v[pl.ds(dch * _L, _L)]
                    for k in range(_L):
                        cnt = cnt + jnp.where(bcast(d_vec, k) < t_vec, one, zero)
                i1 = jnp.clip(cnt, 1, T - 1)
                i0 = i1 - 1
                even = jnp.where(jnp.bitwise_and(i0, 1) == 0, i0, i1)
                ev_v[pl.ds(ch * _L, _L)] = even
                od_v[pl.ds(ch * _L, _L)] = i0 + i1 - even
            pltpu.sync_copy(ev_v, e_hbm.at[pl.ds(b * Ttp, Ttp)])
            pltpu.sync_copy(od_v, o_hbm.at[pl.ds(b * Ttp, Ttp)])

        @pl.when(wid < B)
        def _():
            series(lr_hbm, nl, Tl, lre_hbm, lro_hbm, wid)

        @pl.when(jnp.logical_and(wid >= 8, wid < 8 + B))
        def _():
            series(hr_hbm, nh, Th, hre_hbm, hro_hbm, wid - 8)

    lre, lro, hre, hro = pl.kernel(
        body,
        out_type=[jax.ShapeDtypeStruct((B * Ttp,), jnp.int32)] * 4,
        mesh=mesh,
        scratch_types=[
            pltpu.VMEM((max(Tlp, Thp),), jnp.int32),
            pltpu.VMEM((Ttp,), jnp.int32),
            pltpu.VMEM((Ttp,), jnp.int32),
            pltpu.VMEM((Ttp,), jnp.int32),
        ],
    )(lr_flat, hr_flat, tgt_pad)
    return (lre.reshape(B, Ttp), lro.reshape(B, Ttp),
            hre.reshape(B, Ttp), hro.reshape(B, Ttp))


def _fuse_kernel(lre_p, lro_p, hre_p, hro_p, lr_doy_p, hr_doy_p, tgt_p,
                 lre_ref, lro_ref, hre_ref, hro_ref, m_ref,
                 out_lr_ref, out_hr_ref):
    b = pl.program_id(0)
    t = pl.program_id(1)
    tf = tgt_p[t].astype(jnp.float32)

    def weights(doy_p, e_p, o_p):
        e = e_p[b, t]
        o = o_p[b, t]
        i0 = jnp.minimum(e, o)
        i1 = jnp.maximum(e, o)
        d0 = doy_p[b, i0].astype(jnp.float32)
        d1 = doy_p[b, i1].astype(jnp.float32)
        denom = jnp.where(d1 - d0 == 0.0, 1.0, d1 - d0)
        w = jnp.clip((tf - d0) / denom, 0.0, 1.0)
        c = e < o
        return jnp.where(c, 1.0 - w, w), jnp.where(c, w, 1.0 - w)

    whe, who = weights(hr_doy_p, hre_p, hro_p)
    out_hr_ref[0, 0] = hre_ref[0, 0] * whe + hro_ref[0, 0] * who

    wle, wlo = weights(lr_doy_p, lre_p, lro_p)
    lr = lre_ref[0, 0] * wle + lro_ref[0, 0] * wlo
    m = m_ref[...]
    a = lax.dot_general(lr, m, (((1,), (1,)), ((), ())),
                        preferred_element_type=jnp.float32)
    out = lax.dot_general(a, m, (((1,), (1,)), ((), ())),
                          preferred_element_type=jnp.float32)
    out_lr_ref[0, 0] = out


def kernel(lr_data, hr_data, lr_doy, hr_doy, target_doy):
    B, Tl, C, H, W = lr_data.shape
    _, Th, _, Hh, Wh = hr_data.shape
    Tt = target_doy.shape[0]
    Hout, Wout = Hh, Wh
    lre, lro, hre, hro = _sc_indices(lr_doy, hr_doy, target_doy)

    m = jnp.asarray(_resize_matrix(H, Hout))

    grid_spec = pltpu.PrefetchScalarGridSpec(
        num_scalar_prefetch=7,
        grid=(B, Tt),
        in_specs=[
            pl.BlockSpec((1, 1, C, H, W),
                         lambda b, t, le, lo, he, ho, *_: (b, le[b, t], 0, 0, 0)),
            pl.BlockSpec((1, 1, C, H, W),
                         lambda b, t, le, lo, he, ho, *_: (b, lo[b, t], 0, 0, 0)),
            pl.BlockSpec((1, 1, C, Hh, Wh),
                         lambda b, t, le, lo, he, ho, *_: (b, he[b, t], 0, 0, 0)),
            pl.BlockSpec((1, 1, C, Hh, Wh),
                         lambda b, t, le, lo, he, ho, *_: (b, ho[b, t], 0, 0, 0)),
            pl.BlockSpec((Hout, H), lambda *_: (0, 0)),
        ],
        out_specs=[
            pl.BlockSpec((1, 1, C, Hout, Wout), lambda b, t, *_: (b, t, 0, 0, 0)),
            pl.BlockSpec((1, 1, C, Hh, Wh), lambda b, t, *_: (b, t, 0, 0, 0)),
        ],
    )
    out_lr, out_hr = pl.pallas_call(
        _fuse_kernel,
        grid_spec=grid_spec,
        out_shape=[
            jax.ShapeDtypeStruct((B, Tt, C, Hout, Wout), jnp.float32),
            jax.ShapeDtypeStruct((B, Tt, C, Hh, Wh), jnp.float32),
        ],
    )(lre, lro, hre, hro, lr_doy, hr_doy, target_doy,
      lr_data, lr_data, hr_data, hr_data, m)

    return (out_lr, out_hr)


# speedup vs baseline: 1.0493x; 1.0493x over previous
import functools

import numpy as np
import jax
import jax.numpy as jnp
from jax import lax
from jax.experimental import pallas as pl
from jax.experimental.pallas import tpu as pltpu
from jax.experimental.pallas import tpu_sc as plsc


def _resize_matrix(in_size: int, out_size: int) -> np.ndarray:
    sample_f = (np.arange(out_size) + 0.5) * (in_size / out_size) - 0.5
    x = np.abs(sample_f[:, None] - np.arange(in_size)[None, :])
    w = np.maximum(0.0, 1.0 - x)
    w = w / w.sum(axis=1, keepdims=True)
    return w.astype(np.float32)


_L = 16


def _sc_indices(lr_pad, hr_pad, tgt_pad, Tl, Th):
    B, Tlp = lr_pad.shape
    Thp = hr_pad.shape[1]
    Ttp = tgt_pad.shape[0]
    lr_flat = lr_pad.reshape(-1)
    hr_flat = hr_pad.reshape(-1)
    nl, nh, nt = Tlp // _L, Thp // _L, Ttp // _L

    mesh = plsc.VectorSubcoreMesh(core_axis_name="c", subcore_axis_name="s")
    nc = mesh.num_cores

    def body(lr_hbm, hr_hbm, tgt_hbm, idx_hbm, d_v, tgt_v, eo_v, sem_d, sem_t):
        wid = lax.axis_index("s") * nc + lax.axis_index("c")

        def bcast(vec, k):
            idx = jnp.full((_L, 1), k, jnp.int32)
            return lax.gather(
                vec, idx,
                lax.GatherDimensionNumbers(offset_dims=(),
                                           collapsed_slice_dims=(0,),
                                           start_index_map=(0,)),
                slice_sizes=(1,),
                mode=lax.GatherScatterMode.PROMISE_IN_BOUNDS)

        def series(d_hbm, nchunk, T, b, col0):
            cd = pltpu.make_async_copy(
                d_hbm.at[pl.ds(b * nchunk * _L, nchunk * _L)],
                d_v.at[pl.ds(0, nchunk * _L)], sem_d)
            ct = pltpu.make_async_copy(tgt_hbm, tgt_v, sem_t)
            cd.start()
            ct.start()
            cd.wait()
            ct.wait()
            one = jnp.full((_L,), 1, jnp.int32)
            zero = jnp.full((_L,), 0, jnp.int32)
            for ch in range(nt):
                t_vec = tgt_v[pl.ds(ch * _L, _L)]
                cnt = zero
                for dch in range(nchunk):
                    d_vec = d_v[pl.ds(dch * _L, _L)]
                    for k in range(_L):
                        cnt = cnt + jnp.where(bcast(d_vec, k) < t_vec, one, zero)
                i1 = jnp.clip(cnt, 1, T - 1)
                i0 = i1 - 1
                even = jnp.where(jnp.bitwise_and(i0, 1) == 0, i0, i1)
                eo_v[pl.ds(ch * _L, _L)] = even
                eo_v[pl.ds(Ttp + ch * _L, _L)] = i0 + i1 - even
            pltpu.sync_copy(eo_v, idx_hbm.at[pl.ds(b * 4 * Ttp + col0, 2 * Ttp)])

        @pl.when(wid < B)
        def _():
            series(lr_hbm, nl, Tl, wid, 0)

        @pl.when(jnp.logical_and(wid >= 8, wid < 8 + B))
        def _():
            series(hr_hbm, nh, Th, wid - 8, 2 * Ttp)

    return pl.kernel(
        body,
        out_type=jax.ShapeDtypeStruct((B * 4 * Ttp,), jnp.int32),
        mesh=mesh,
        scratch_types=[
            pltpu.VMEM((max(Tlp, Thp),), jnp.int32),
            pltpu.VMEM((Ttp,), jnp.int32),
            pltpu.VMEM((2 * Ttp,), jnp.int32),
            pltpu.SemaphoreType.DMA,
            pltpu.SemaphoreType.DMA,
        ],
    )(lr_flat, hr_flat, tgt_pad)


def _fuse_kernel(idx_p, lr_doy_p, hr_doy_p, tgt_p,
                 lre_ref, lro_ref, hre_ref, hro_ref, m_ref,
                 out_lr_ref, out_hr_ref):
    b = pl.program_id(0)
    t = pl.program_id(1)
    tf = tgt_p[t].astype(jnp.float32)

    def weights(doy_p, col0):
        e = idx_p[b * 128 + col0 + t]
        o = idx_p[b * 128 + col0 + 32 + t]
        i0 = jnp.minimum(e, o)
        i1 = jnp.maximum(e, o)
        d0 = doy_p[b, i0].astype(jnp.float32)
        d1 = doy_p[b, i1].astype(jnp.float32)
        denom = jnp.where(d1 - d0 == 0.0, 1.0, d1 - d0)
        w = jnp.clip((tf - d0) / denom, 0.0, 1.0)
        c = e < o
        return jnp.where(c, 1.0 - w, w), jnp.where(c, w, 1.0 - w)

    whe, who = weights(hr_doy_p, 64)
    out_hr_ref[0, 0] = hre_ref[0, 0] * whe + hro_ref[0, 0] * who

    wle, wlo = weights(lr_doy_p, 0)
    lr = lre_ref[0, 0] * wle + lro_ref[0, 0] * wlo
    m = m_ref[...]
    a = lax.dot_general(lr, m, (((1,), (1,)), ((), ())),
                        preferred_element_type=jnp.float32)
    out = lax.dot_general(a, m, (((1,), (1,)), ((), ())),
                          preferred_element_type=jnp.float32)
    out_lr_ref[0, 0] = out


def kernel(lr_data, hr_data, lr_doy, hr_doy, target_doy):
    B, Tl, C, H, W = lr_data.shape
    _, Th, _, Hh, Wh = hr_data.shape
    Tt = target_doy.shape[0]
    Hout, Wout = Hh, Wh
    Tlp = ((Tl + _L - 1) // _L) * _L
    Thp = ((Th + _L - 1) // _L) * _L
    Ttp = ((Tt + _L - 1) // _L) * _L
    big = jnp.int32(2 ** 30)
    lr_pad = jnp.concatenate(
        [lr_doy, jnp.full((B, Tlp - Tl), big, jnp.int32)], axis=1)
    hr_pad = jnp.concatenate(
        [hr_doy, jnp.full((B, Thp - Th), big, jnp.int32)], axis=1)
    tgt_pad = jnp.concatenate(
        [target_doy, jnp.zeros((Ttp - Tt,), jnp.int32)])

    idx = _sc_indices(lr_pad, hr_pad, tgt_pad, Tl, Th)

    m = jnp.asarray(_resize_matrix(H, Hout))

    grid_spec = pltpu.PrefetchScalarGridSpec(
        num_scalar_prefetch=4,
        grid=(B, Tt),
        in_specs=[
            pl.BlockSpec((1, 1, C, H, W),
                         lambda b, t, ix, *_: (b, ix[b * 128 + t], 0, 0, 0)),
            pl.BlockSpec((1, 1, C, H, W),
                         lambda b, t, ix, *_: (b, ix[b * 128 + 32 + t], 0, 0, 0)),
            pl.BlockSpec((1, 1, C, Hh, Wh),
                         lambda b, t, ix, *_: (b, ix[b * 128 + 64 + t], 0, 0, 0)),
            pl.BlockSpec((1, 1, C, Hh, Wh),
                         lambda b, t, ix, *_: (b, ix[b * 128 + 96 + t], 0, 0, 0)),
            pl.BlockSpec((Hout, H), lambda *_: (0, 0)),
        ],
        out_specs=[
            pl.BlockSpec((1, 1, C, Hout, Wout), lambda b, t, *_: (b, t, 0, 0, 0)),
            pl.BlockSpec((1, 1, C, Hh, Wh), lambda b, t, *_: (b, t, 0, 0, 0)),
        ],
    )
    out_lr, out_hr = pl.pallas_call(
        _fuse_kernel,
        grid_spec=grid_spec,
        out_shape=[
            jax.ShapeDtypeStruct((B, Tt, C, Hout, Wout), jnp.float32),
            jax.ShapeDtypeStruct((B, Tt, C, Hh, Wh), jnp.float32),
        ],
    )(idx, lr_doy, hr_doy, target_doy,
      lr_data, lr_data, hr_data, hr_data, m)

    return (out_lr, out_hr)


# speedup vs baseline: 1.0549x; 1.0054x over previous
import functools

import numpy as np
import jax
import jax.numpy as jnp
from jax import lax
from jax.experimental import pallas as pl
from jax.experimental.pallas import tpu as pltpu
from jax.experimental.pallas import tpu_sc as plsc


def _resize_matrix(in_size: int, out_size: int) -> np.ndarray:
    sample_f = (np.arange(out_size) + 0.5) * (in_size / out_size) - 0.5
    x = np.abs(sample_f[:, None] - np.arange(in_size)[None, :])
    w = np.maximum(0.0, 1.0 - x)
    w = w / w.sum(axis=1, keepdims=True)
    return w.astype(np.float32)


_L = 16


def _sc_indices(dat, B, Tl, Th, Tlp, Thp, Ttp):
    nl, nh, nt = Tlp // _L, Thp // _L, Ttp // _L
    hr0 = B * Tlp
    tg0 = B * (Tlp + Thp)

    mesh = plsc.VectorSubcoreMesh(core_axis_name="c", subcore_axis_name="s")
    nc = mesh.num_cores

    def body(dat_hbm, idx_hbm, d_v, tgt_v, eo_v, sem_d, sem_t):
        wid = lax.axis_index("s") * nc + lax.axis_index("c")

        def bcast(vec, k):
            idx = jnp.full((_L, 1), k, jnp.int32)
            return lax.gather(
                vec, idx,
                lax.GatherDimensionNumbers(offset_dims=(),
                                           collapsed_slice_dims=(0,),
                                           start_index_map=(0,)),
                slice_sizes=(1,),
                mode=lax.GatherScatterMode.PROMISE_IN_BOUNDS)

        def series(d0, nchunk, T, b, col0):
            cd = pltpu.make_async_copy(
                dat_hbm.at[pl.ds(d0 + b * nchunk * _L, nchunk * _L)],
                d_v.at[pl.ds(0, nchunk * _L)], sem_d)
            ct = pltpu.make_async_copy(dat_hbm.at[pl.ds(tg0, Ttp)], tgt_v, sem_t)
            cd.start()
            ct.start()
            cd.wait()
            ct.wait()
            one = jnp.full((_L,), 1, jnp.int32)
            zero = jnp.full((_L,), 0, jnp.int32)
            for ch in range(nt):
                t_vec = tgt_v[pl.ds(ch * _L, _L)]
                cnt = zero
                for dch in range(nchunk):
                    d_vec = d_v[pl.ds(dch * _L, _L)]
                    for k in range(_L):
                        cnt = cnt + jnp.where(bcast(d_vec, k) < t_vec, one, zero)
                i1 = jnp.clip(cnt, 1, T - 1)
                i0 = i1 - 1
                even = jnp.where(jnp.bitwise_and(i0, 1) == 0, i0, i1)
                eo_v[pl.ds(ch * _L, _L)] = even
                eo_v[pl.ds(Ttp + ch * _L, _L)] = i0 + i1 - even
            pltpu.sync_copy(eo_v, idx_hbm.at[pl.ds(b * 4 * Ttp + col0, 2 * Ttp)])

        @pl.when(wid < B)
        def _():
            series(0, nl, Tl, wid, 0)

        @pl.when(jnp.logical_and(wid >= 8, wid < 8 + B))
        def _():
            series(hr0, nh, Th, wid - 8, 2 * Ttp)

    return pl.kernel(
        body,
        out_type=jax.ShapeDtypeStruct((B * 4 * Ttp,), jnp.int32),
        mesh=mesh,
        scratch_types=[
            pltpu.VMEM((max(Tlp, Thp),), jnp.int32),
            pltpu.VMEM((Ttp,), jnp.int32),
            pltpu.VMEM((2 * Ttp,), jnp.int32),
            pltpu.SemaphoreType.DMA,
            pltpu.SemaphoreType.DMA,
        ],
    )(dat)


def _fuse_kernel(idx_p, lr_doy_p, hr_doy_p, tgt_p,
                 lre_ref, lro_ref, hre_ref, hro_ref, m_ref,
                 out_lr_ref, out_hr_ref):
    b = pl.program_id(0)
    t = pl.program_id(1)
    tf = tgt_p[t].astype(jnp.float32)

    def weights(doy_p, col0):
        e = idx_p[b * 128 + col0 + t]
        o = idx_p[b * 128 + col0 + 32 + t]
        i0 = jnp.minimum(e, o)
        i1 = jnp.maximum(e, o)
        d0 = doy_p[b, i0].astype(jnp.float32)
        d1 = doy_p[b, i1].astype(jnp.float32)
        denom = jnp.where(d1 - d0 == 0.0, 1.0, d1 - d0)
        w = jnp.clip((tf - d0) / denom, 0.0, 1.0)
        c = e < o
        return jnp.where(c, 1.0 - w, w), jnp.where(c, w, 1.0 - w)

    whe, who = weights(hr_doy_p, 64)
    out_hr_ref[0, 0] = hre_ref[0, 0] * whe + hro_ref[0, 0] * who

    wle, wlo = weights(lr_doy_p, 0)
    lr = lre_ref[0, 0] * wle + lro_ref[0, 0] * wlo
    m = m_ref[...]
    a = lax.dot_general(lr, m, (((1,), (1,)), ((), ())),
                        preferred_element_type=jnp.float32)
    out = lax.dot_general(a, m, (((1,), (1,)), ((), ())),
                          preferred_element_type=jnp.float32)
    out_lr_ref[0, 0] = out


def kernel(lr_data, hr_data, lr_doy, hr_doy, target_doy):
    B, Tl, C, H, W = lr_data.shape
    _, Th, _, Hh, Wh = hr_data.shape
    Tt = target_doy.shape[0]
    Hout, Wout = Hh, Wh
    Tlp = ((Tl + _L - 1) // _L) * _L
    Thp = ((Th + _L - 1) // _L) * _L
    Ttp = ((Tt + _L - 1) // _L) * _L
    big = jnp.int32(2 ** 30)
    lr_pad = jnp.concatenate(
        [lr_doy, jnp.full((B, Tlp - Tl), big, jnp.int32)], axis=1)
    hr_pad = jnp.concatenate(
        [hr_doy, jnp.full((B, Thp - Th), big, jnp.int32)], axis=1)
    tgt_pad = jnp.concatenate(
        [target_doy, jnp.zeros((Ttp - Tt,), jnp.int32)])
    dat = jnp.concatenate(
        [lr_pad.reshape(-1), hr_pad.reshape(-1), tgt_pad])

    idx = _sc_indices(dat, B, Tl, Th, Tlp, Thp, Ttp)

    m = jnp.asarray(_resize_matrix(H, Hout))

    grid_spec = pltpu.PrefetchScalarGridSpec(
        num_scalar_prefetch=4,
        grid=(B, Tt),
        in_specs=[
            pl.BlockSpec((1, 1, C, H, W),
                         lambda b, t, ix, *_: (b, ix[b * 128 + t], 0, 0, 0)),
            pl.BlockSpec((1, 1, C, H, W),
                         lambda b, t, ix, *_: (b, ix[b * 128 + 32 + t], 0, 0, 0)),
            pl.BlockSpec((1, 1, C, Hh, Wh),
                         lambda b, t, ix, *_: (b, ix[b * 128 + 64 + t], 0, 0, 0)),
            pl.BlockSpec((1, 1, C, Hh, Wh),
                         lambda b, t, ix, *_: (b, ix[b * 128 + 96 + t], 0, 0, 0)),
            pl.BlockSpec((Hout, H), lambda *_: (0, 0)),
        ],
        out_specs=[
            pl.BlockSpec((1, 1, C, Hout, Wout), lambda b, t, *_: (b, t, 0, 0, 0)),
            pl.BlockSpec((1, 1, C, Hh, Wh), lambda b, t, *_: (b, t, 0, 0, 0)),
        ],
    )
    out_lr, out_hr = pl.pallas_call(
        _fuse_kernel,
        grid_spec=grid_spec,
        out_shape=[
            jax.ShapeDtypeStruct((B, Tt, C, Hout, Wout), jnp.float32),
            jax.ShapeDtypeStruct((B, Tt, C, Hh, Wh), jnp.float32),
        ],
    )(idx, lr_doy, hr_doy, target_doy,
      lr_data, lr_data, hr_data, hr_data, m)

    return (out_lr, out_hr)


# speedup vs baseline: 1.0583x; 1.0032x over previous
import functools

import numpy as np
import jax
import jax.numpy as jnp
from jax import lax
from jax.experimental import pallas as pl
from jax.experimental.pallas import tpu as pltpu
from jax.experimental.pallas import tpu_sc as plsc


def _resize_matrix(in_size: int, out_size: int) -> np.ndarray:
    sample_f = (np.arange(out_size) + 0.5) * (in_size / out_size) - 0.5
    x = np.abs(sample_f[:, None] - np.arange(in_size)[None, :])
    w = np.maximum(0.0, 1.0 - x)
    w = w / w.sum(axis=1, keepdims=True)
    return w.astype(np.float32)


_L = 16


def _sc_indices(dat, B, Tl, Th, Tlp, Thp, Ttp):
    nl, nh, nt = Tlp // _L, Thp // _L, Ttp // _L
    hr0 = B * Tlp
    tg0 = B * (Tlp + Thp)

    mesh = plsc.VectorSubcoreMesh(core_axis_name="c", subcore_axis_name="s")
    nc = mesh.num_cores

    def body(dat_hbm, idx_hbm, d_v, tgt_v, eo_v, sem_d, sem_t):
        wid = lax.axis_index("s") * nc + lax.axis_index("c")

        def bcast(vec, k):
            idx = jnp.full((_L, 1), k, jnp.int32)
            return lax.gather(
                vec, idx,
                lax.GatherDimensionNumbers(offset_dims=(),
                                           collapsed_slice_dims=(0,),
                                           start_index_map=(0,)),
                slice_sizes=(1,),
                mode=lax.GatherScatterMode.PROMISE_IN_BOUNDS)

        def series(d0, nchunk, T, b, col0):
            cd = pltpu.make_async_copy(
                dat_hbm.at[pl.ds(d0 + b * nchunk * _L, nchunk * _L)],
                d_v.at[pl.ds(0, nchunk * _L)], sem_d)
            ct = pltpu.make_async_copy(dat_hbm.at[pl.ds(tg0, Ttp)], tgt_v, sem_t)
            cd.start()
            ct.start()
            cd.wait()
            ct.wait()
            one = jnp.full((_L,), 1, jnp.int32)
            zero = jnp.full((_L,), 0, jnp.int32)
            for ch in range(nt):
                t_vec = tgt_v[pl.ds(ch * _L, _L)]
                cnt = zero
                for dch in range(nchunk):
                    d_vec = d_v[pl.ds(dch * _L, _L)]
                    for k in range(min(_L, T - dch * _L)):
                        cnt = cnt + jnp.where(bcast(d_vec, k) < t_vec, one, zero)
                i1 = jnp.clip(cnt, 1, T - 1)
                i0 = i1 - 1
                even = jnp.where(jnp.bitwise_and(i0, 1) == 0, i0, i1)
                eo_v[pl.ds(ch * _L, _L)] = even
                eo_v[pl.ds(Ttp + ch * _L, _L)] = i0 + i1 - even
            pltpu.sync_copy(eo_v, idx_hbm.at[pl.ds(b * 4 * Ttp + col0, 2 * Ttp)])

        @pl.when(wid < B)
        def _():
            series(0, nl, Tl, wid, 0)

        @pl.when(jnp.logical_and(wid >= 8, wid < 8 + B))
        def _():
            series(hr0, nh, Th, wid - 8, 2 * Ttp)

    return pl.kernel(
        body,
        out_type=jax.ShapeDtypeStruct((B * 4 * Ttp,), jnp.int32),
        mesh=mesh,
        scratch_types=[
            pltpu.VMEM((max(Tlp, Thp),), jnp.int32),
            pltpu.VMEM((Ttp,), jnp.int32),
            pltpu.VMEM((2 * Ttp,), jnp.int32),
            pltpu.SemaphoreType.DMA,
            pltpu.SemaphoreType.DMA,
        ],
    )(dat)


def _fuse_kernel(idx_p, lr_doy_p, hr_doy_p, tgt_p,
                 lre_ref, lro_ref, hre_ref, hro_ref, m_ref,
                 out_lr_ref, out_hr_ref):
    b = pl.program_id(0)
    t = pl.program_id(1)
    tf = tgt_p[t].astype(jnp.float32)

    def weights(doy_p, col0):
        e = idx_p[b * 128 + col0 + t]
        o = idx_p[b * 128 + col0 + 32 + t]
        i0 = jnp.minimum(e, o)
        i1 = jnp.maximum(e, o)
        d0 = doy_p[b, i0].astype(jnp.float32)
        d1 = doy_p[b, i1].astype(jnp.float32)
        denom = jnp.where(d1 - d0 == 0.0, 1.0, d1 - d0)
        w = jnp.clip((tf - d0) / denom, 0.0, 1.0)
        c = e < o
        return jnp.where(c, 1.0 - w, w), jnp.where(c, w, 1.0 - w)

    whe, who = weights(hr_doy_p, 64)
    out_hr_ref[0, 0] = hre_ref[0, 0] * whe + hro_ref[0, 0] * who

    wle, wlo = weights(lr_doy_p, 0)
    lr = lre_ref[0, 0] * wle + lro_ref[0, 0] * wlo
    m = m_ref[...]
    a = lax.dot_general(lr, m, (((1,), (1,)), ((), ())),
                        preferred_element_type=jnp.float32)
    out = lax.dot_general(a, m, (((1,), (1,)), ((), ())),
                          preferred_element_type=jnp.float32)
    out_lr_ref[0, 0] = out


def kernel(lr_data, hr_data, lr_doy, hr_doy, target_doy):
    B, Tl, C, H, W = lr_data.shape
    _, Th, _, Hh, Wh = hr_data.shape
    Tt = target_doy.shape[0]
    Hout, Wout = Hh, Wh
    Tlp = ((Tl + _L - 1) // _L) * _L
    Thp = ((Th + _L - 1) // _L) * _L
    Ttp = ((Tt + _L - 1) // _L) * _L
    zero = jnp.int32(0)
    lr_pad = jnp.concatenate(
        [lr_doy, jnp.full((B, Tlp - Tl), zero, jnp.int32)], axis=1)
    hr_pad = jnp.concatenate(
        [hr_doy, jnp.full((B, Thp - Th), zero, jnp.int32)], axis=1)
    tgt_pad = jnp.concatenate(
        [target_doy, jnp.zeros((Ttp - Tt,), jnp.int32)])
    dat = jnp.concatenate(
        [lr_pad.reshape(-1), hr_pad.reshape(-1), tgt_pad])

    idx = _sc_indices(dat, B, Tl, Th, Tlp, Thp, Ttp)

    m = jnp.asarray(_resize_matrix(H, Hout))

    grid_spec = pltpu.PrefetchScalarGridSpec(
        num_scalar_prefetch=4,
        grid=(B, Tt),
        in_specs=[
            pl.BlockSpec((1, 1, C, H, W),
                         lambda b, t, ix, *_: (b, ix[b * 128 + t], 0, 0, 0)),
            pl.BlockSpec((1, 1, C, H, W),
                         lambda b, t, ix, *_: (b, ix[b * 128 + 32 + t], 0, 0, 0)),
            pl.BlockSpec((1, 1, C, Hh, Wh),
                         lambda b, t, ix, *_: (b, ix[b * 128 + 64 + t], 0, 0, 0)),
            pl.BlockSpec((1, 1, C, Hh, Wh),
                         lambda b, t, ix, *_: (b, ix[b * 128 + 96 + t], 0, 0, 0)),
            pl.BlockSpec((Hout, H), lambda *_: (0, 0)),
        ],
        out_specs=[
            pl.BlockSpec((1, 1, C, Hout, Wout), lambda b, t, *_: (b, t, 0, 0, 0)),
            pl.BlockSpec((1, 1, C, Hh, Wh), lambda b, t, *_: (b, t, 0, 0, 0)),
        ],
    )
    out_lr, out_hr = pl.pallas_call(
        _fuse_kernel,
        grid_spec=grid_spec,
        out_shape=[
            jax.ShapeDtypeStruct((B, Tt, C, Hout, Wout), jnp.float32),
            jax.ShapeDtypeStruct((B, Tt, C, Hh, Wh), jnp.float32),
        ],
    )(idx, lr_doy, hr_doy, target_doy,
      lr_data, lr_data, hr_data, hr_data, m)

    return (out_lr, out_hr)
